# baseline (speedup 1.0000x reference)
import functools

import jax
import jax.numpy as jnp
from jax import lax
from jax.experimental import pallas as pl
from jax.experimental.pallas import tpu as pltpu
from jax.experimental.pallas import tpu_sc as plsc

N = 50048
E = 800768
H = 256
NC = 2
NS = 16
NT = NC * NS
EA = E + N
EPT = EA // NT
BB = 128
EPTP = ((EPT + BB - 1) // BB) * BB
NB = EPTP // BB
RPT = N // NS
SB = 16
NSTG = NB // SB
ZR = 136
NZ = RPT // ZR
NBLK = N // 64
RB = 1088
GRID = N // RB
AB = RB // 64
F32 = jnp.float32


def _make_bag(nchunk, cw):
  mesh = plsc.VectorSubcoreMesh(core_axis_name="c", subcore_axis_name="s")
  out_shape = ((NC, N, 2, 128) if cw == 32 else (NC, nchunk, N, cw))

  @functools.partial(
      pl.kernel,
      out_type=jax.ShapeDtypeStruct(out_shape, F32),
      mesh=mesh,
      compiler_params=pltpu.CompilerParams(use_tc_tiling_on_sc=False),
      scratch_types=[
          pltpu.VMEM((SB, BB), jnp.int32),
          pltpu.VMEM((SB, BB), jnp.int32),
          pltpu.VMEM((BB, cw), F32),
          pltpu.VMEM((BB, cw), F32),
          pltpu.VMEM((BB, cw), F32),
          pltpu.VMEM((BB, cw), F32),
          pltpu.VMEM((ZR, cw), F32),
          pltpu.VMEM_SHARED((N + 16, cw), F32),
          pltpu.SemaphoreType.DMA,
          pltpu.SemaphoreType.DMA,
          pltpu.SemaphoreType.DMA,
          pltpu.SemaphoreType.DMA,
          pltpu.SemaphoreType.DMA,
          pltpu.SemaphoreType.DMA,
          pltpu.SemaphoreType.DMA,
          pltpu.SemaphoreType.DMA,
      ],
  )
  def bag(table_hbm, src_hbm, dst_hbm, out_hbm,
          src_v, dst_v, buf0, buf1, buf2, buf3, zer, acc,
          sem0, sem1, sem2, sem3, sem4, sem5, sem6, sem7):
    bufs = (buf0, buf1, buf2, buf3)
    sems = (sem0, sem1, sem2, sem3)
    ssems = (sem4, sem5, sem6, sem7)
    cid = lax.axis_index("c")
    sid = lax.axis_index("s")
    tid = cid * NS + sid
    row0 = sid * RPT

    def zfill(i, carry):
      for off in range(0, cw, 16):
        zer[i, pl.ds(off, 16)] = jnp.zeros((16,), F32)
      return carry
    lax.fori_loop(0, ZR, zfill, 0)

    def zero_own_rows():
      for z in range(NZ):
        pltpu.async_copy(zer, acc.at[pl.ds(row0 + z * ZR, ZR)], sem0)
      for z in range(NZ):
        pltpu.make_async_copy(zer, acc.at[pl.ds(row0, ZR)], sem0).wait()

    zero_own_rows()
    plsc.subcore_barrier()

    for c in range(nchunk):
      def stage(stg, carry):
        pltpu.sync_copy(src_hbm.at[c, tid, pl.ds(stg * SB, SB)], src_v)
        pltpu.sync_copy(dst_hbm.at[tid, pl.ds(stg * SB, SB)], dst_v)

        for b in range(4):
          pltpu.async_copy(table_hbm.at[src_v.at[b]], bufs[b], sems[b])

        def batch(k, carry2):
          for b in range(4):
            j = 4 * k + b
            pltpu.make_async_copy(
                table_hbm.at[src_v.at[j]], bufs[b], sems[b]).wait()
            pltpu.async_copy(bufs[b], acc.at[dst_v.at[j]], ssems[b],
                             add=True)
          for b in range(4):
            j = 4 * k + b
            pltpu.make_async_copy(
                bufs[b], acc.at[dst_v.at[j]], ssems[b]).wait()

            @pl.when(k < SB // 4 - 1)
            def _():
              pltpu.async_copy(
                  table_hbm.at[src_v.at[j + 4]], bufs[b], sems[b])
          return carry2
        lax.fori_loop(0, SB // 4, batch, 0)
        return carry
      lax.fori_loop(0, NSTG, stage, 0)

      plsc.subcore_barrier()
      if cw == 32:
        pltpu.sync_copy(
            acc.at[pl.ds(row0, RPT)],
            out_hbm.at[cid, pl.ds(row0, RPT), c // 4,
                       pl.ds(32 * (c % 4), 32)])
      else:
        pltpu.sync_copy(acc.at[pl.ds(row0, RPT)],
                        out_hbm.at[cid, c, pl.ds(row0, RPT)])
      if c + 1 < nchunk:
        zero_own_rows()
      plsc.subcore_barrier()

  return bag


_bag16 = _make_bag(1, 16)
_bag32 = _make_bag(8, 32)


def _tca_body(degp_ref, x16_ref, d16_ref, xs_ref):
  deg = degp_ref[0, 0] + degp_ref[1, 0]
  d = lax.rsqrt(deg)
  d16_ref[...] = d
  xs_ref[...] = d * x16_ref[...]


def _tc1_body(p_ref, d_ref, w1_ref, b1_ref, w2_ref, out_ref):
  d = d_ref[...]
  s = d * (p_ref[0, 0] + p_ref[1, 0])
  h = jnp.dot(s, w1_ref[...], preferred_element_type=F32) + b1_ref[...]
  h = jnp.maximum(h, 0.0)
  g = d[:, :1] * jnp.dot(h, w2_ref[...], preferred_element_type=F32)
  for c in range(8):
    out_ref[c] = g[:, c * 32:(c + 1) * 32]


def _tcmid_body(p_ref, d_ref, b_ref, w_ref, *rest, act, out16):
  if act:
    act_ref, wa_ref, out_ref = rest
  else:
    out_ref, = rest
  s = (p_ref[0] + p_ref[1]).reshape(RB, 256)
  d = d_ref[...]
  h = jnp.maximum(d[:, :1] * s + b_ref[...], 0.0)
  g = jnp.dot(h, w_ref[...], preferred_element_type=F32)
  if act:
    ab = jnp.dot(act_ref[:, 0, :], wa_ref[...], preferred_element_type=F32)
    g = g + jnp.broadcast_to(ab[:, None, :], (AB, 64, H)).reshape(RB, H)
  g = d[:, :1] * g
  if out16:
    out_ref[...] = g
  else:
    for c in range(8):
      out_ref[c] = g[:, c * 32:(c + 1) * 32]


def _tc6_body(p_ref, d_ref, b_ref, out_ref):
  s = d_ref[...] * (p_ref[0, 0] + p_ref[1, 0])
  out_ref[...] = jnp.maximum(s + b_ref[...], 0.0)


def _spec16(i):
  return (i, 0)


_B16 = pl.BlockSpec((RB, 16), _spec16)
_BP16 = pl.BlockSpec((2, 1, RB, 16), lambda i: (0, 0, i, 0))
_BP128 = pl.BlockSpec((2, 2 * RB, 128), lambda i: (0, i, 0))
_BG32 = pl.BlockSpec((8, RB, 32), lambda i: (0, i, 0))
_BFULL = lambda shape: pl.BlockSpec(shape, lambda i: tuple(0 for _ in shape))

_tca = pl.pallas_call(
    _tca_body, grid=(GRID,),
    in_specs=[_BP16, _B16],
    out_specs=[_B16, _B16],
    out_shape=[jax.ShapeDtypeStruct((N, 16), F32),
               jax.ShapeDtypeStruct((N, 16), F32)])

_tc1 = pl.pallas_call(
    _tc1_body, grid=(GRID,),
    in_specs=[_BP16, _B16, _BFULL((16, H)), _BFULL((1, H)),
              _BFULL((H, H))],
    out_specs=_BG32,
    out_shape=jax.ShapeDtypeStruct((8, N, 32), F32))

_tcmid = pl.pallas_call(
    functools.partial(_tcmid_body, act=False, out16=False), grid=(GRID,),
    in_specs=[_BP128, _B16, _BFULL((1, H)), _BFULL((H, H))],
    out_specs=_BG32,
    out_shape=jax.ShapeDtypeStruct((8, N, 32), F32))

_tcmid_act = pl.pallas_call(
    functools.partial(_tcmid_body, act=True, out16=False), grid=(GRID,),
    in_specs=[_BP128, _B16, _BFULL((1, H)), _BFULL((H, H)),
              pl.BlockSpec((AB, 1, 8), lambda i: (i, 0, 0)),
              _BFULL((8, H))],
    out_specs=_BG32,
    out_shape=jax.ShapeDtypeStruct((8, N, 32), F32))

_tcmid16 = pl.pallas_call(
    functools.partial(_tcmid_body, act=False, out16=True), grid=(GRID,),
    in_specs=[_BP128, _B16, _BFULL((1, H)), _BFULL((H, 16))],
    out_specs=_B16,
    out_shape=jax.ShapeDtypeStruct((N, 16), F32))

_tc6 = pl.pallas_call(
    _tc6_body, grid=(GRID,),
    in_specs=[_BP16, _B16, _BFULL((1, 16))],
    out_specs=_B16,
    out_shape=jax.ShapeDtypeStruct((N, 16), F32))


def kernel(x, edge_index, action, W1, b1, W2, b2, W3, b3, W4, b4, W5, b5,
           W6, b6):
  loop = jnp.arange(N, dtype=jnp.int32)
  src = jnp.concatenate([edge_index[0], loop])
  dst = jnp.concatenate([edge_index[1], loop])

  srcp = jnp.pad(src.reshape(NT, EPT), ((0, 0), (0, EPTP - EPT)))
  srcp = srcp.reshape(NT, NB, BB)
  dstp = jnp.pad(dst.reshape(NT, EPT), ((0, 0), (0, EPTP - EPT)),
                 constant_values=N)
  dstp = dstp.reshape(NT, NB, BB)
  src1 = srcp[None]
  src8 = (srcp[None] + (jnp.arange(8, dtype=jnp.int32) * N)[:, None, None,
                                                            None])

  x16 = jnp.pad(x, ((0, 0), (0, 13)))
  w1p = jnp.pad(W1, ((0, 13), (0, 0)))
  w6p = jnp.pad(W6, ((0, 0), (0, 13)))
  b6p = jnp.pad(b6, (0, 13))
  actp = jnp.pad(action, ((0, 0), (0, 3))).reshape(NBLK, 1, 8)
  wap = jnp.pad(W4[H:], ((0, 3), (0, 0)))
  w4h = W4[:H]
  b1r, b2r, b3r, b4r, b5r = (b.reshape(1, H) for b in (b1, b2, b3, b4, b5))
  b6r = b6p.reshape(1, 16)

  ones16 = jnp.ones((N, 16), F32)
  degp = _bag16(ones16, src1, dstp)
  d16, xs = _tca(degp, x16)

  s1p = _bag16(xs, src1, dstp)
  g2 = _tc1(s1p, d16, w1p, b1r, W2)

  p2 = _bag32(g2.reshape(8 * N, 32), src8, dstp).reshape(NC, 2 * N, 128)
  g3 = _tcmid(p2, d16, b2r, W3)

  p3 = _bag32(g3.reshape(8 * N, 32), src8, dstp).reshape(NC, 2 * N, 128)
  g4 = _tcmid_act(p3, d16, b3r, w4h, actp, wap)

  p4 = _bag32(g4.reshape(8 * N, 32), src8, dstp).reshape(NC, 2 * N, 128)
  g5 = _tcmid(p4, d16, b4r, W5)

  p5 = _bag32(g5.reshape(8 * N, 32), src8, dstp).reshape(NC, 2 * N, 128)
  g6 = _tcmid16(p5, d16, b5r, w6p)

  p6 = _bag16(g6, src1, dstp)
  out16 = _tc6(p6, d16, b6r)
  return out16[:, :3]


# speedup vs baseline: 1.0280x; 1.0280x over previous
import functools

import jax
import jax.numpy as jnp
from jax import lax
from jax.experimental import pallas as pl
from jax.experimental.pallas import tpu as pltpu
from jax.experimental.pallas import tpu_sc as plsc

N = 50048
E = 800768
H = 256
NC = 2
NS = 16
NT = NC * NS
EA = E + N
EPT = EA // NT
BB = 128
EPTP = ((EPT + BB - 1) // BB) * BB
NB = EPTP // BB
RPT = N // NS
SB = 16
NSTG = NB // SB
ZR = 136
NZ = RPT // ZR
NBLK = N // 64
RB = 1088
GRID = N // RB
AB = RB // 64
F32 = jnp.float32


def _make_bag(nchunk, cw):
  mesh = plsc.VectorSubcoreMesh(core_axis_name="c", subcore_axis_name="s")
  out_shape = ((NC, N, 2, 128) if cw == 32 else (NC, nchunk, N, cw))

  @functools.partial(
      pl.kernel,
      out_type=jax.ShapeDtypeStruct(out_shape, F32),
      mesh=mesh,
      compiler_params=pltpu.CompilerParams(use_tc_tiling_on_sc=False),
      scratch_types=[
          pltpu.VMEM((SB, BB), jnp.int32),
          pltpu.VMEM((SB, BB), jnp.int32),
          pltpu.VMEM((BB, cw), F32),
          pltpu.VMEM((BB, cw), F32),
          pltpu.VMEM((BB, cw), F32),
          pltpu.VMEM((BB, cw), F32),
          pltpu.VMEM((ZR, cw), F32),
          pltpu.VMEM_SHARED((N + 16, cw), F32),
          pltpu.SemaphoreType.DMA,
          pltpu.SemaphoreType.DMA,
          pltpu.SemaphoreType.DMA,
          pltpu.SemaphoreType.DMA,
      ],
  )
  def bag(table_hbm, src_hbm, dst_hbm, out_hbm,
          src_v, dst_v, buf0, buf1, buf2, buf3, zer, acc,
          sem0, sem1, sem2, sem3):
    bufs = (buf0, buf1, buf2, buf3)
    sems = (sem0, sem1, sem2, sem3)
    cid = lax.axis_index("c")
    sid = lax.axis_index("s")
    tid = cid * NS + sid
    row0 = sid * RPT

    def zfill(i, carry):
      for off in range(0, cw, 16):
        zer[i, pl.ds(off, 16)] = jnp.zeros((16,), F32)
      return carry
    lax.fori_loop(0, ZR, zfill, 0)

    def zero_own_rows():
      for z in range(NZ):
        pltpu.async_copy(zer, acc.at[pl.ds(row0 + z * ZR, ZR)], sem0)
      for z in range(NZ):
        pltpu.make_async_copy(zer, acc.at[pl.ds(row0, ZR)], sem0).wait()

    zero_own_rows()
    plsc.subcore_barrier()

    for c in range(nchunk):
      def stage(stg, carry):
        pltpu.sync_copy(src_hbm.at[c, tid, pl.ds(stg * SB, SB)], src_v)
        pltpu.sync_copy(dst_hbm.at[tid, pl.ds(stg * SB, SB)], dst_v)

        for b in range(4):
          pltpu.async_copy(table_hbm.at[src_v.at[b]], bufs[b], sems[b])

        def batch(k, carry2):
          for b in range(4):
            j = 4 * k + b
            pltpu.make_async_copy(
                table_hbm.at[src_v.at[j]], bufs[b], sems[b]).wait()
            pltpu.sync_copy(bufs[b], acc.at[dst_v.at[j]], add=True)

            @pl.when(k < SB // 4 - 1)
            def _():
              pltpu.async_copy(
                  table_hbm.at[src_v.at[j + 4]], bufs[b], sems[b])
          return carry2
        lax.fori_loop(0, SB // 4, batch, 0)
        return carry
      lax.fori_loop(0, NSTG, stage, 0)

      plsc.subcore_barrier()
      if cw == 32:
        pltpu.sync_copy(
            acc.at[pl.ds(row0, RPT)],
            out_hbm.at[cid, pl.ds(row0, RPT), c // 4,
                       pl.ds(32 * (c % 4), 32)])
      else:
        pltpu.sync_copy(acc.at[pl.ds(row0, RPT)],
                        out_hbm.at[cid, c, pl.ds(row0, RPT)])
      if c + 1 < nchunk:
        zero_own_rows()
      plsc.subcore_barrier()

  return bag


_bag16 = _make_bag(1, 16)
_bag32 = _make_bag(8, 32)


def _tca_body(degp_ref, x16_ref, d16_ref, xs_ref):
  deg = degp_ref[0, 0] + degp_ref[1, 0]
  d = lax.rsqrt(deg)
  d16_ref[...] = d
  xs_ref[...] = d * x16_ref[...]


def _tc1_body(p_ref, d_ref, w1_ref, b1_ref, w2_ref, out_ref):
  d = d_ref[...]
  s = d * (p_ref[0, 0] + p_ref[1, 0])
  h = jnp.dot(s, w1_ref[...], preferred_element_type=F32) + b1_ref[...]
  h = jnp.maximum(h, 0.0)
  g = d[:, :1] * jnp.dot(h, w2_ref[...], preferred_element_type=F32)
  for c in range(8):
    out_ref[c] = g[:, c * 32:(c + 1) * 32]


def _tcmid_body(p_ref, d_ref, b_ref, w_ref, *rest, act, out16):
  if act:
    act_ref, wa_ref, out_ref = rest
  else:
    out_ref, = rest
  s = (p_ref[0] + p_ref[1]).reshape(RB, 256)
  d = d_ref[...]
  h = jnp.maximum(d[:, :1] * s + b_ref[...], 0.0)
  g = jnp.dot(h, w_ref[...], preferred_element_type=F32)
  if act:
    ab = jnp.dot(act_ref[:, 0, :], wa_ref[...], preferred_element_type=F32)
    g = g + jnp.broadcast_to(ab[:, None, :], (AB, 64, H)).reshape(RB, H)
  g = d[:, :1] * g
  if out16:
    out_ref[...] = g
  else:
    for c in range(8):
      out_ref[c] = g[:, c * 32:(c + 1) * 32]


def _tc6_body(p_ref, d_ref, b_ref, out_ref):
  s = d_ref[...] * (p_ref[0, 0] + p_ref[1, 0])
  out_ref[...] = jnp.maximum(s + b_ref[...], 0.0)


def _spec16(i):
  return (i, 0)


_B16 = pl.BlockSpec((RB, 16), _spec16)
_BP16 = pl.BlockSpec((2, 1, RB, 16), lambda i: (0, 0, i, 0))
_BP128 = pl.BlockSpec((2, 2 * RB, 128), lambda i: (0, i, 0))
_BG32 = pl.BlockSpec((8, RB, 32), lambda i: (0, i, 0))
_BFULL = lambda shape: pl.BlockSpec(shape, lambda i: tuple(0 for _ in shape))

_tca = pl.pallas_call(
    _tca_body, grid=(GRID,),
    in_specs=[_BP16, _B16],
    out_specs=[_B16, _B16],
    out_shape=[jax.ShapeDtypeStruct((N, 16), F32),
               jax.ShapeDtypeStruct((N, 16), F32)])

_tc1 = pl.pallas_call(
    _tc1_body, grid=(GRID,),
    in_specs=[_BP16, _B16, _BFULL((16, H)), _BFULL((1, H)),
              _BFULL((H, H))],
    out_specs=_BG32,
    out_shape=jax.ShapeDtypeStruct((8, N, 32), F32))

_tcmid = pl.pallas_call(
    functools.partial(_tcmid_body, act=False, out16=False), grid=(GRID,),
    in_specs=[_BP128, _B16, _BFULL((1, H)), _BFULL((H, H))],
    out_specs=_BG32,
    out_shape=jax.ShapeDtypeStruct((8, N, 32), F32))

_tcmid_act = pl.pallas_call(
    functools.partial(_tcmid_body, act=True, out16=False), grid=(GRID,),
    in_specs=[_BP128, _B16, _BFULL((1, H)), _BFULL((H, H)),
              pl.BlockSpec((AB, 1, 8), lambda i: (i, 0, 0)),
              _BFULL((8, H))],
    out_specs=_BG32,
    out_shape=jax.ShapeDtypeStruct((8, N, 32), F32))

_tcmid16 = pl.pallas_call(
    functools.partial(_tcmid_body, act=False, out16=True), grid=(GRID,),
    in_specs=[_BP128, _B16, _BFULL((1, H)), _BFULL((H, 16))],
    out_specs=_B16,
    out_shape=jax.ShapeDtypeStruct((N, 16), F32))

_tc6 = pl.pallas_call(
    _tc6_body, grid=(GRID,),
    in_specs=[_BP16, _B16, _BFULL((1, 16))],
    out_specs=_B16,
    out_shape=jax.ShapeDtypeStruct((N, 16), F32))


def kernel(x, edge_index, action, W1, b1, W2, b2, W3, b3, W4, b4, W5, b5,
           W6, b6):
  loop = jnp.arange(N, dtype=jnp.int32)
  src = jnp.concatenate([edge_index[0], loop])
  dst = jnp.concatenate([edge_index[1], loop])

  srcp = jnp.pad(src.reshape(NT, EPT), ((0, 0), (0, EPTP - EPT)))
  srcp = srcp.reshape(NT, NB, BB)
  dstp = jnp.pad(dst.reshape(NT, EPT), ((0, 0), (0, EPTP - EPT)),
                 constant_values=N)
  dstp = dstp.reshape(NT, NB, BB)
  src1 = srcp[None]
  src8 = (srcp[None] + (jnp.arange(8, dtype=jnp.int32) * N)[:, None, None,
                                                            None])

  x16 = jnp.pad(x, ((0, 0), (0, 13)))
  w1p = jnp.pad(W1, ((0, 13), (0, 0)))
  w6p = jnp.pad(W6, ((0, 0), (0, 13)))
  b6p = jnp.pad(b6, (0, 13))
  actp = jnp.pad(action, ((0, 0), (0, 3))).reshape(NBLK, 1, 8)
  wap = jnp.pad(W4[H:], ((0, 3), (0, 0)))
  w4h = W4[:H]
  b1r, b2r, b3r, b4r, b5r = (b.reshape(1, H) for b in (b1, b2, b3, b4, b5))
  b6r = b6p.reshape(1, 16)

  ones16 = jnp.ones((N, 16), F32)
  degp = _bag16(ones16, src1, dstp)
  d16, xs = _tca(degp, x16)

  s1p = _bag16(xs, src1, dstp)
  g2 = _tc1(s1p, d16, w1p, b1r, W2)

  p2 = _bag32(g2.reshape(8 * N, 32), src8, dstp).reshape(NC, 2 * N, 128)
  g3 = _tcmid(p2, d16, b2r, W3)

  p3 = _bag32(g3.reshape(8 * N, 32), src8, dstp).reshape(NC, 2 * N, 128)
  g4 = _tcmid_act(p3, d16, b3r, w4h, actp, wap)

  p4 = _bag32(g4.reshape(8 * N, 32), src8, dstp).reshape(NC, 2 * N, 128)
  g5 = _tcmid(p4, d16, b4r, W5)

  p5 = _bag32(g5.reshape(8 * N, 32), src8, dstp).reshape(NC, 2 * N, 128)
  g6 = _tcmid16(p5, d16, b5r, w6p)

  p6 = _bag16(g6, src1, dstp)
  out16 = _tc6(p6, d16, b6r)
  return out16[:, :3]


# speedup vs baseline: 1.1163x; 1.0859x over previous
import functools

import jax
import jax.numpy as jnp
from jax import lax
from jax.experimental import pallas as pl
from jax.experimental.pallas import tpu as pltpu
from jax.experimental.pallas import tpu_sc as plsc

N = 50048
E = 800768
H = 256
NC = 2
NS = 16
NT = NC * NS
EA = E + N
EPT = EA // NT
BB = 128
EPTP = ((EPT + BB - 1) // BB) * BB
NB = EPTP // BB
RPT = N // NS
SB = 16
NSTG = NB // SB
ZR = 136
NZ = RPT // ZR
NBLK = N // 64
RB = 1088
GRID = N // RB
AB = RB // 64
F32 = jnp.float32


def _make_bag(nchunk, cw):
  mesh = plsc.VectorSubcoreMesh(core_axis_name="c", subcore_axis_name="s")
  out_shape = ((NC, N, 2, 128) if cw == 32 else (NC, nchunk, N, cw))

  @functools.partial(
      pl.kernel,
      out_type=jax.ShapeDtypeStruct(out_shape, F32),
      mesh=mesh,
      compiler_params=pltpu.CompilerParams(use_tc_tiling_on_sc=False),
      scratch_types=[
          pltpu.VMEM((2, SB, BB), jnp.int32),
          pltpu.VMEM((2, SB, BB), jnp.int32),
          pltpu.VMEM((BB, cw), F32),
          pltpu.VMEM((BB, cw), F32),
          pltpu.VMEM((BB, cw), F32),
          pltpu.VMEM((BB, cw), F32),
          pltpu.VMEM((ZR, cw), F32),
          pltpu.VMEM_SHARED((N + 16, cw), F32),
          pltpu.SemaphoreType.DMA,
          pltpu.SemaphoreType.DMA,
          pltpu.SemaphoreType.DMA,
          pltpu.SemaphoreType.DMA,
          pltpu.SemaphoreType.DMA,
          pltpu.SemaphoreType.DMA,
      ],
  )
  def bag(table_hbm, src_hbm, dst_hbm, out_hbm,
          src_v2, dst_v2, buf0, buf1, buf2, buf3, zer, acc,
          sem0, sem1, sem2, sem3, isem0, isem1):
    bufs = (buf0, buf1, buf2, buf3)
    sems = (sem0, sem1, sem2, sem3)
    isems = (isem0, isem1)
    cid = lax.axis_index("c")
    sid = lax.axis_index("s")
    tid = cid * NS + sid
    row0 = sid * RPT

    def idx_start(c, stg, slot):
      pltpu.async_copy(src_hbm.at[c, tid, pl.ds(stg * SB, SB)],
                       src_v2.at[slot], isems[slot])
      pltpu.async_copy(dst_hbm.at[tid, pl.ds(stg * SB, SB)],
                       dst_v2.at[slot], isems[slot])

    def idx_wait(c, stg, slot):
      pltpu.make_async_copy(src_hbm.at[c, tid, pl.ds(stg * SB, SB)],
                            src_v2.at[slot], isems[slot]).wait()
      pltpu.make_async_copy(dst_hbm.at[tid, pl.ds(stg * SB, SB)],
                            dst_v2.at[slot], isems[slot]).wait()

    def zfill(i, carry):
      for off in range(0, cw, 16):
        zer[i, pl.ds(off, 16)] = jnp.zeros((16,), F32)
      return carry
    lax.fori_loop(0, ZR, zfill, 0)

    def zero_own_rows():
      for z in range(NZ):
        pltpu.async_copy(zer, acc.at[pl.ds(row0 + z * ZR, ZR)], sem0)
      for z in range(NZ):
        pltpu.make_async_copy(zer, acc.at[pl.ds(row0, ZR)], sem0).wait()

    zero_own_rows()
    plsc.subcore_barrier()

    for c in range(nchunk):
      def process(stg, slot):
        idx_wait(c, stg, slot)
        src_v = src_v2.at[slot]
        dst_v = dst_v2.at[slot]

        for b in range(4):
          pltpu.async_copy(table_hbm.at[src_v.at[b]], bufs[b], sems[b])

        def batch(k, carry2):
          for b in range(4):
            j = 4 * k + b
            pltpu.make_async_copy(
                table_hbm.at[src_v.at[j]], bufs[b], sems[b]).wait()
            pltpu.sync_copy(bufs[b], acc.at[dst_v.at[j]], add=True)

            @pl.when(k < SB // 4 - 1)
            def _():
              pltpu.async_copy(
                  table_hbm.at[src_v.at[j + 4]], bufs[b], sems[b])
          return carry2
        lax.fori_loop(0, SB // 4, batch, 0)

      idx_start(c, 0, 0)

      def spair(kk, carry):
        stg = 2 * kk
        idx_start(c, stg + 1, 1)
        process(stg, 0)
        idx_start(c, stg + 2, 0)
        process(stg + 1, 1)
        return carry
      lax.fori_loop(0, (NSTG - 1) // 2, spair, 0)
      process(NSTG - 1, 0)

      plsc.subcore_barrier()
      if cw == 32:
        pltpu.sync_copy(
            acc.at[pl.ds(row0, RPT)],
            out_hbm.at[cid, pl.ds(row0, RPT), c // 4,
                       pl.ds(32 * (c % 4), 32)])
      else:
        pltpu.sync_copy(acc.at[pl.ds(row0, RPT)],
                        out_hbm.at[cid, c, pl.ds(row0, RPT)])
      if c + 1 < nchunk:
        zero_own_rows()
      plsc.subcore_barrier()

  return bag


_bag16 = _make_bag(1, 16)
_bag32 = _make_bag(8, 32)


def _tca_body(degp_ref, x16_ref, d16_ref, xs_ref):
  deg = degp_ref[0, 0] + degp_ref[1, 0]
  d = lax.rsqrt(deg)
  d16_ref[...] = d
  xs_ref[...] = d * x16_ref[...]


def _tc1_body(p_ref, d_ref, w1_ref, b1_ref, w2_ref, out_ref):
  d = d_ref[...]
  s = d * (p_ref[0, 0] + p_ref[1, 0])
  h = jnp.dot(s, w1_ref[...], preferred_element_type=F32) + b1_ref[...]
  h = jnp.maximum(h, 0.0)
  g = d[:, :1] * jnp.dot(h, w2_ref[...], preferred_element_type=F32)
  for c in range(8):
    out_ref[c] = g[:, c * 32:(c + 1) * 32]


def _tcmid_body(p_ref, d_ref, b_ref, w_ref, *rest, act, out16):
  if act:
    act_ref, wa_ref, out_ref = rest
  else:
    out_ref, = rest
  s = (p_ref[0] + p_ref[1]).reshape(RB, 256)
  d = d_ref[...]
  h = jnp.maximum(d[:, :1] * s + b_ref[...], 0.0)
  g = jnp.dot(h, w_ref[...], preferred_element_type=F32)
  if act:
    ab = jnp.dot(act_ref[:, 0, :], wa_ref[...], preferred_element_type=F32)
    g = g + jnp.broadcast_to(ab[:, None, :], (AB, 64, H)).reshape(RB, H)
  g = d[:, :1] * g
  if out16:
    out_ref[...] = g
  else:
    for c in range(8):
      out_ref[c] = g[:, c * 32:(c + 1) * 32]


def _tc6_body(p_ref, d_ref, b_ref, out_ref):
  s = d_ref[...] * (p_ref[0, 0] + p_ref[1, 0])
  out_ref[...] = jnp.maximum(s + b_ref[...], 0.0)


def _spec16(i):
  return (i, 0)


_B16 = pl.BlockSpec((RB, 16), _spec16)
_BP16 = pl.BlockSpec((2, 1, RB, 16), lambda i: (0, 0, i, 0))
_BP128 = pl.BlockSpec((2, 2 * RB, 128), lambda i: (0, i, 0))
_BG32 = pl.BlockSpec((8, RB, 32), lambda i: (0, i, 0))
_BFULL = lambda shape: pl.BlockSpec(shape, lambda i: tuple(0 for _ in shape))

_tca = pl.pallas_call(
    _tca_body, grid=(GRID,),
    in_specs=[_BP16, _B16],
    out_specs=[_B16, _B16],
    out_shape=[jax.ShapeDtypeStruct((N, 16), F32),
               jax.ShapeDtypeStruct((N, 16), F32)])

_tc1 = pl.pallas_call(
    _tc1_body, grid=(GRID,),
    in_specs=[_BP16, _B16, _BFULL((16, H)), _BFULL((1, H)),
              _BFULL((H, H))],
    out_specs=_BG32,
    out_shape=jax.ShapeDtypeStruct((8, N, 32), F32))

_tcmid = pl.pallas_call(
    functools.partial(_tcmid_body, act=False, out16=False), grid=(GRID,),
    in_specs=[_BP128, _B16, _BFULL((1, H)), _BFULL((H, H))],
    out_specs=_BG32,
    out_shape=jax.ShapeDtypeStruct((8, N, 32), F32))

_tcmid_act = pl.pallas_call(
    functools.partial(_tcmid_body, act=True, out16=False), grid=(GRID,),
    in_specs=[_BP128, _B16, _BFULL((1, H)), _BFULL((H, H)),
              pl.BlockSpec((AB, 1, 8), lambda i: (i, 0, 0)),
              _BFULL((8, H))],
    out_specs=_BG32,
    out_shape=jax.ShapeDtypeStruct((8, N, 32), F32))

_tcmid16 = pl.pallas_call(
    functools.partial(_tcmid_body, act=False, out16=True), grid=(GRID,),
    in_specs=[_BP128, _B16, _BFULL((1, H)), _BFULL((H, 16))],
    out_specs=_B16,
    out_shape=jax.ShapeDtypeStruct((N, 16), F32))

_tc6 = pl.pallas_call(
    _tc6_body, grid=(GRID,),
    in_specs=[_BP16, _B16, _BFULL((1, 16))],
    out_specs=_B16,
    out_shape=jax.ShapeDtypeStruct((N, 16), F32))


def kernel(x, edge_index, action, W1, b1, W2, b2, W3, b3, W4, b4, W5, b5,
           W6, b6):
  loop = jnp.arange(N, dtype=jnp.int32)
  src = jnp.concatenate([edge_index[0], loop])
  dst = jnp.concatenate([edge_index[1], loop])

  srcp = jnp.pad(src.reshape(NT, EPT), ((0, 0), (0, EPTP - EPT)))
  srcp = srcp.reshape(NT, NB, BB)
  dstp = jnp.pad(dst.reshape(NT, EPT), ((0, 0), (0, EPTP - EPT)),
                 constant_values=N)
  dstp = dstp.reshape(NT, NB, BB)
  src1 = srcp[None]
  src8 = (srcp[None] + (jnp.arange(8, dtype=jnp.int32) * N)[:, None, None,
                                                            None])

  x16 = jnp.pad(x, ((0, 0), (0, 13)))
  w1p = jnp.pad(W1, ((0, 13), (0, 0)))
  w6p = jnp.pad(W6, ((0, 0), (0, 13)))
  b6p = jnp.pad(b6, (0, 13))
  actp = jnp.pad(action, ((0, 0), (0, 3))).reshape(NBLK, 1, 8)
  wap = jnp.pad(W4[H:], ((0, 3), (0, 0)))
  w4h = W4[:H]
  b1r, b2r, b3r, b4r, b5r = (b.reshape(1, H) for b in (b1, b2, b3, b4, b5))
  b6r = b6p.reshape(1, 16)

  ones16 = jnp.ones((N, 16), F32)
  degp = _bag16(ones16, src1, dstp)
  d16, xs = _tca(degp, x16)

  s1p = _bag16(xs, src1, dstp)
  g2 = _tc1(s1p, d16, w1p, b1r, W2)

  p2 = _bag32(g2.reshape(8 * N, 32), src8, dstp).reshape(NC, 2 * N, 128)
  g3 = _tcmid(p2, d16, b2r, W3)

  p3 = _bag32(g3.reshape(8 * N, 32), src8, dstp).reshape(NC, 2 * N, 128)
  g4 = _tcmid_act(p3, d16, b3r, w4h, actp, wap)

  p4 = _bag32(g4.reshape(8 * N, 32), src8, dstp).reshape(NC, 2 * N, 128)
  g5 = _tcmid(p4, d16, b4r, W5)

  p5 = _bag32(g5.reshape(8 * N, 32), src8, dstp).reshape(NC, 2 * N, 128)
  g6 = _tcmid16(p5, d16, b5r, w6p)

  p6 = _bag16(g6, src1, dstp)
  out16 = _tc6(p6, d16, b6r)
  return out16[:, :3]


# speedup vs baseline: 1.1231x; 1.0061x over previous
import functools

import jax
import jax.numpy as jnp
from jax import lax
from jax.experimental import pallas as pl
from jax.experimental.pallas import tpu as pltpu
from jax.experimental.pallas import tpu_sc as plsc

N = 50048
E = 800768
H = 256
NC = 2
NS = 16
NT = NC * NS
EA = E + N
EPT = EA // NT
BB = 128
EPTP = ((EPT + BB - 1) // BB) * BB
NB = EPTP // BB
RPT = N // NS
SB = 16
NSTG = NB // SB
ZR = 136
NZ = RPT // ZR
NBLK = N // 64
RB = 1088
GRID = N // RB
AB = RB // 64
F32 = jnp.float32


def _make_bag(nchunk, cw):
  mesh = plsc.VectorSubcoreMesh(core_axis_name="c", subcore_axis_name="s")
  out_shape = ((NC, N, 2, 128) if cw == 32 else (NC, nchunk, N, cw))

  @functools.partial(
      pl.kernel,
      out_type=jax.ShapeDtypeStruct(out_shape, F32),
      mesh=mesh,
      compiler_params=pltpu.CompilerParams(use_tc_tiling_on_sc=False),
      scratch_types=[
          pltpu.VMEM((2, SB, BB), jnp.int32),
          pltpu.VMEM((2, SB, BB), jnp.int32),
          pltpu.VMEM((BB, cw), F32),
          pltpu.VMEM((BB, cw), F32),
          pltpu.VMEM((BB, cw), F32),
          pltpu.VMEM((BB, cw), F32),
          pltpu.VMEM((ZR, cw), F32),
          pltpu.VMEM_SHARED((N + 16, cw), F32),
          pltpu.SemaphoreType.DMA,
          pltpu.SemaphoreType.DMA,
          pltpu.SemaphoreType.DMA,
          pltpu.SemaphoreType.DMA,
          pltpu.SemaphoreType.DMA,
          pltpu.SemaphoreType.DMA,
      ],
  )
  def bag(table_hbm, src_hbm, dst_hbm, out_hbm,
          src_v2, dst_v2, buf0, buf1, buf2, buf3, zer, acc,
          sem0, sem1, sem2, sem3, isem0, isem1):
    bufs = (buf0, buf1, buf2, buf3)
    sems = (sem0, sem1, sem2, sem3)
    isems = (isem0, isem1)
    cid = lax.axis_index("c")
    sid = lax.axis_index("s")
    tid = cid * NS + sid
    row0 = sid * RPT

    def idx_start(c, stg, slot):
      pltpu.async_copy(src_hbm.at[c, tid, pl.ds(stg * SB, SB)],
                       src_v2.at[slot], isems[slot])
      pltpu.async_copy(dst_hbm.at[tid, pl.ds(stg * SB, SB)],
                       dst_v2.at[slot], isems[slot])

    def idx_wait(c, stg, slot):
      pltpu.make_async_copy(src_hbm.at[c, tid, pl.ds(stg * SB, SB)],
                            src_v2.at[slot], isems[slot]).wait()
      pltpu.make_async_copy(dst_hbm.at[tid, pl.ds(stg * SB, SB)],
                            dst_v2.at[slot], isems[slot]).wait()

    def zfill(i, carry):
      for off in range(0, cw, 16):
        zer[i, pl.ds(off, 16)] = jnp.zeros((16,), F32)
      return carry
    lax.fori_loop(0, ZR, zfill, 0)

    def zero_own_rows():
      for z in range(NZ):
        pltpu.async_copy(zer, acc.at[pl.ds(row0 + z * ZR, ZR)], sem0)
      for z in range(NZ):
        pltpu.make_async_copy(zer, acc.at[pl.ds(row0, ZR)], sem0).wait()

    zero_own_rows()
    plsc.subcore_barrier()

    for c in range(nchunk):
      def process(stg, slot):
        idx_wait(c, stg, slot)
        src_v = src_v2.at[slot]
        dst_v = dst_v2.at[slot]

        for b in range(4):
          pltpu.async_copy(table_hbm.at[src_v.at[b]], bufs[b], sems[b])

        def batch(k, carry2):
          for b in range(4):
            j = 4 * k + b
            pltpu.make_async_copy(
                table_hbm.at[src_v.at[j]], bufs[b], sems[b]).wait()
            pltpu.sync_copy(bufs[b], acc.at[dst_v.at[j]], add=True)

            @pl.when(k < SB // 4 - 1)
            def _():
              pltpu.async_copy(
                  table_hbm.at[src_v.at[j + 4]], bufs[b], sems[b])
          return carry2
        lax.fori_loop(0, SB // 4, batch, 0)

      idx_start(c, 0, 0)

      def spair(kk, carry):
        stg = 2 * kk
        idx_start(c, stg + 1, 1)
        process(stg, 0)
        idx_start(c, stg + 2, 0)
        process(stg + 1, 1)
        return carry
      lax.fori_loop(0, (NSTG - 1) // 2, spair, 0)
      process(NSTG - 1, 0)

      plsc.subcore_barrier()
      for z in range(NZ):
        blk = pl.ds(row0 + z * ZR, ZR)
        if cw == 32:
          pltpu.sync_copy(acc.at[blk],
                          out_hbm.at[cid, blk, c // 4,
                                     pl.ds(32 * (c % 4), 32)])
        else:
          pltpu.sync_copy(acc.at[blk], out_hbm.at[cid, c, blk])
        if c + 1 < nchunk:
          pltpu.async_copy(zer, acc.at[blk], isem0)
      if c + 1 < nchunk:
        for z in range(NZ):
          pltpu.make_async_copy(zer, acc.at[pl.ds(row0, ZR)], isem0).wait()
      plsc.subcore_barrier()

  return bag


_bag16 = _make_bag(1, 16)
_bag32 = _make_bag(8, 32)


def _tca_body(degp_ref, x16_ref, d16_ref, xs_ref):
  deg = degp_ref[0, 0] + degp_ref[1, 0]
  d = lax.rsqrt(deg)
  d16_ref[...] = d
  xs_ref[...] = d * x16_ref[...]


def _tc1_body(p_ref, d_ref, w1_ref, b1_ref, w2_ref, out_ref):
  d = d_ref[...]
  s = d * (p_ref[0, 0] + p_ref[1, 0])
  h = jnp.dot(s, w1_ref[...], preferred_element_type=F32) + b1_ref[...]
  h = jnp.maximum(h, 0.0)
  g = d[:, :1] * jnp.dot(h, w2_ref[...], preferred_element_type=F32)
  for c in range(8):
    out_ref[c] = g[:, c * 32:(c + 1) * 32]


def _tcmid_body(p_ref, d_ref, b_ref, w_ref, *rest, act, out16):
  if act:
    act_ref, wa_ref, out_ref = rest
  else:
    out_ref, = rest
  s = (p_ref[0] + p_ref[1]).reshape(RB, 256)
  d = d_ref[...]
  h = jnp.maximum(d[:, :1] * s + b_ref[...], 0.0)
  g = jnp.dot(h, w_ref[...], preferred_element_type=F32)
  if act:
    ab = jnp.dot(act_ref[:, 0, :], wa_ref[...], preferred_element_type=F32)
    g = g + jnp.broadcast_to(ab[:, None, :], (AB, 64, H)).reshape(RB, H)
  g = d[:, :1] * g
  if out16:
    out_ref[...] = g
  else:
    for c in range(8):
      out_ref[c] = g[:, c * 32:(c + 1) * 32]


def _tc6_body(p_ref, d_ref, b_ref, out_ref):
  s = d_ref[...] * (p_ref[0, 0] + p_ref[1, 0])
  out_ref[...] = jnp.maximum(s + b_ref[...], 0.0)


def _spec16(i):
  return (i, 0)


_B16 = pl.BlockSpec((RB, 16), _spec16)
_BP16 = pl.BlockSpec((2, 1, RB, 16), lambda i: (0, 0, i, 0))
_BP128 = pl.BlockSpec((2, 2 * RB, 128), lambda i: (0, i, 0))
_BG32 = pl.BlockSpec((8, RB, 32), lambda i: (0, i, 0))
_BFULL = lambda shape: pl.BlockSpec(shape, lambda i: tuple(0 for _ in shape))

_tca = pl.pallas_call(
    _tca_body, grid=(GRID,),
    in_specs=[_BP16, _B16],
    out_specs=[_B16, _B16],
    out_shape=[jax.ShapeDtypeStruct((N, 16), F32),
               jax.ShapeDtypeStruct((N, 16), F32)])

_tc1 = pl.pallas_call(
    _tc1_body, grid=(GRID,),
    in_specs=[_BP16, _B16, _BFULL((16, H)), _BFULL((1, H)),
              _BFULL((H, H))],
    out_specs=_BG32,
    out_shape=jax.ShapeDtypeStruct((8, N, 32), F32))

_tcmid = pl.pallas_call(
    functools.partial(_tcmid_body, act=False, out16=False), grid=(GRID,),
    in_specs=[_BP128, _B16, _BFULL((1, H)), _BFULL((H, H))],
    out_specs=_BG32,
    out_shape=jax.ShapeDtypeStruct((8, N, 32), F32))

_tcmid_act = pl.pallas_call(
    functools.partial(_tcmid_body, act=True, out16=False), grid=(GRID,),
    in_specs=[_BP128, _B16, _BFULL((1, H)), _BFULL((H, H)),
              pl.BlockSpec((AB, 1, 8), lambda i: (i, 0, 0)),
              _BFULL((8, H))],
    out_specs=_BG32,
    out_shape=jax.ShapeDtypeStruct((8, N, 32), F32))

_tcmid16 = pl.pallas_call(
    functools.partial(_tcmid_body, act=False, out16=True), grid=(GRID,),
    in_specs=[_BP128, _B16, _BFULL((1, H)), _BFULL((H, 16))],
    out_specs=_B16,
    out_shape=jax.ShapeDtypeStruct((N, 16), F32))

_tc6 = pl.pallas_call(
    _tc6_body, grid=(GRID,),
    in_specs=[_BP16, _B16, _BFULL((1, 16))],
    out_specs=_B16,
    out_shape=jax.ShapeDtypeStruct((N, 16), F32))


def kernel(x, edge_index, action, W1, b1, W2, b2, W3, b3, W4, b4, W5, b5,
           W6, b6):
  loop = jnp.arange(N, dtype=jnp.int32)
  src = jnp.concatenate([edge_index[0], loop])
  dst = jnp.concatenate([edge_index[1], loop])

  srcp = jnp.pad(src.reshape(NT, EPT), ((0, 0), (0, EPTP - EPT)))
  srcp = srcp.reshape(NT, NB, BB)
  dstp = jnp.pad(dst.reshape(NT, EPT), ((0, 0), (0, EPTP - EPT)),
                 constant_values=N)
  dstp = dstp.reshape(NT, NB, BB)
  src1 = srcp[None]
  src8 = (srcp[None] + (jnp.arange(8, dtype=jnp.int32) * N)[:, None, None,
                                                            None])

  x16 = jnp.pad(x, ((0, 0), (0, 13)))
  w1p = jnp.pad(W1, ((0, 13), (0, 0)))
  w6p = jnp.pad(W6, ((0, 0), (0, 13)))
  b6p = jnp.pad(b6, (0, 13))
  actp = jnp.pad(action, ((0, 0), (0, 3))).reshape(NBLK, 1, 8)
  wap = jnp.pad(W4[H:], ((0, 3), (0, 0)))
  w4h = W4[:H]
  b1r, b2r, b3r, b4r, b5r = (b.reshape(1, H) for b in (b1, b2, b3, b4, b5))
  b6r = b6p.reshape(1, 16)

  ones16 = jnp.ones((N, 16), F32)
  degp = _bag16(ones16, src1, dstp)
  d16, xs = _tca(degp, x16)

  s1p = _bag16(xs, src1, dstp)
  g2 = _tc1(s1p, d16, w1p, b1r, W2)

  p2 = _bag32(g2.reshape(8 * N, 32), src8, dstp).reshape(NC, 2 * N, 128)
  g3 = _tcmid(p2, d16, b2r, W3)

  p3 = _bag32(g3.reshape(8 * N, 32), src8, dstp).reshape(NC, 2 * N, 128)
  g4 = _tcmid_act(p3, d16, b3r, w4h, actp, wap)

  p4 = _bag32(g4.reshape(8 * N, 32), src8, dstp).reshape(NC, 2 * N, 128)
  g5 = _tcmid(p4, d16, b4r, W5)

  p5 = _bag32(g5.reshape(8 * N, 32), src8, dstp).reshape(NC, 2 * N, 128)
  g6 = _tcmid16(p5, d16, b5r, w6p)

  p6 = _bag16(g6, src1, dstp)
  out16 = _tc6(p6, d16, b6r)
  return out16[:, :3]
